# Initial kernel scaffold; baseline (speedup 1.0000x reference)
import jax
import jax.numpy as jnp
from jax.experimental import pallas as pl


def kernel(x, edge_index, node_type, edge_type, edge_attr, W_in, nt_emb, et_emb, W_ea, W_msg, a_src, a_dst, W_out_node, W_out_edge):
    raise NotImplementedError("write your pallas kernel here")


# speedup vs baseline: 2.9055x; 2.9055x over previous
import jax
import jax.numpy as jnp
from jax import lax
from jax.experimental import pallas as pl
from jax.experimental.pallas import tpu as pltpu
from jax.experimental.pallas import tpu_sc as plsc

N = 10000
E = 320000
D = 128
H = 128
NT = 6
ET = 4
DE = 4

NP_ = 10240
B1 = 256
NC = 2
NS = 16
HH = H // NC
CH = 64
EPT = 20032
EPAD = NS * EPT
NCHUNK = EPT // CH
VP = NP_ // 2
VD = NP_ // 128
NRN = VP // NS
NRD = VD // NS


def _tc1_body(x_ref, oh_ref, win_ref, nt8_ref, wmsg_ref, a_ref,
              h_ref, hm_ref, hsd_ref):
    xb = x_ref[...]
    h = jnp.maximum(
        jnp.dot(xb, win_ref[...], preferred_element_type=jnp.float32)
        + jnp.dot(oh_ref[...], nt8_ref[...],
                  preferred_element_type=jnp.float32), 0.0)
    h_ref[...] = h
    hm_ref[...] = jnp.dot(h, wmsg_ref[...],
                          preferred_element_type=jnp.float32)
    hsd_ref[...] = jnp.dot(h, a_ref[...],
                           preferred_element_type=jnp.float32)


_tc1 = pl.pallas_call(
    _tc1_body,
    grid=(NP_ // B1,),
    in_specs=[
        pl.BlockSpec((B1, D), lambda i: (i, 0)),
        pl.BlockSpec((B1, 8), lambda i: (i, 0)),
        pl.BlockSpec((D, H), lambda i: (0, 0)),
        pl.BlockSpec((8, H), lambda i: (0, 0)),
        pl.BlockSpec((H, H), lambda i: (0, 0)),
        pl.BlockSpec((H, 8), lambda i: (0, 0)),
    ],
    out_specs=[
        pl.BlockSpec((B1, H), lambda i: (i, 0)),
        pl.BlockSpec((B1, H), lambda i: (i, 0)),
        pl.BlockSpec((B1, 8), lambda i: (i, 0)),
    ],
    out_shape=[
        jax.ShapeDtypeStruct((NP_, H), jnp.float32),
        jax.ShapeDtypeStruct((NP_, H), jnp.float32),
        jax.ShapeDtypeStruct((NP_, 8), jnp.float32),
    ],
)


def _tc2_body(h_ref, num_ref, den_ref, wo_ref, out_ref):
    d = den_ref[...] + 1e-9
    agg = num_ref[...] / d
    h_out = jnp.maximum(h_ref[...] + agg, 0.0)
    out_ref[...] = jnp.dot(h_out, wo_ref[...],
                           preferred_element_type=jnp.float32)


_tc2 = pl.pallas_call(
    _tc2_body,
    grid=(NP_ // B1,),
    in_specs=[
        pl.BlockSpec((B1, H), lambda i: (i, 0)),
        pl.BlockSpec((B1, H), lambda i: (i, 0)),
        pl.BlockSpec((B1, 1), lambda i: (i, 0)),
        pl.BlockSpec((H, 8), lambda i: (0, 0)),
    ],
    out_specs=pl.BlockSpec((B1, 8), lambda i: (i, 0)),
    out_shape=jax.ShapeDtypeStruct((NP_, 8), jnp.float32),
)


def _tc3_body(eo_ref, out_ref):
    v = eo_ref[...]
    out_ref[...] = v[0] + v[1]


_tc3 = pl.pallas_call(
    _tc3_body,
    grid=(1,),
    in_specs=[pl.BlockSpec((NC, EPAD // 128, 128), lambda i: (0, 0, 0))],
    out_specs=pl.BlockSpec((EPAD // 128, 128), lambda i: (0, 0)),
    out_shape=jax.ShapeDtypeStruct((EPAD // 128, 128), jnp.float32),
)


def _sc_body(hm, hs, hd, srcp, dstp, etp, eap, ett2, wea2, woe2, z,
             numer_out, den_out, eo_out,
             hs_v, hd_v, ett_v, wea_v, woe_v,
             src_v, dst_v, et_v, idxn_v, idxd_v, ea_v,
             rows_v, pay_v, denp_v, ex_v, eo_v,
             numer_sh, den_sh, sem):
    c = lax.axis_index("c")
    s = lax.axis_index("s")

    pltpu.sync_copy(hs, hs_v)
    pltpu.sync_copy(hd, hd_v)
    pltpu.sync_copy(ett2.at[c], ett_v)
    pltpu.sync_copy(wea2.at[c], wea_v)
    pltpu.sync_copy(woe2.at[c], woe_v)

    rn0 = s * NRN
    pltpu.sync_copy(z.at[pl.ds(rn0, NRN)], numer_sh.at[pl.ds(rn0, NRN)])

    @pl.when(s == 0)
    def _():
        pltpu.sync_copy(z.at[pl.ds(0, VD)], den_sh.at[pl.ds(0, VD)])

    lane = lax.iota(jnp.int32, 16)
    zeros16 = jnp.zeros((16,), jnp.int32)
    zeros16f = jnp.zeros((16,), jnp.float32)

    def zero_denp(i, _):
        denp_v[i // 8, pl.ds((i % 8) * 16, 16)] = zeros16f
        return 0
    lax.fori_loop(0, CH * 8, zero_denp, 0)

    plsc.subcore_barrier()

    base0 = s * EPT
    coff = c * HH

    def chunk(g, _):
        base = base0 + g * CH
        pltpu.sync_copy(srcp.at[pl.ds(base, CH)], src_v)
        pltpu.sync_copy(dstp.at[pl.ds(base, CH)], dst_v)
        pltpu.sync_copy(etp.at[pl.ds(base, CH)], et_v)
        pltpu.sync_copy(eap.at[pl.ds(base, CH)], ea_v)
        pltpu.async_copy(hm.at[src_v], rows_v, sem).wait()

        def phase_a(i, _):
            sl = pl.ds(i * 16, 16)
            si = src_v[sl]
            di = dst_v[sl]
            idxn_v[sl] = lax.shift_right_logical(di, 1)
            idxd_v[sl] = lax.shift_right_logical(di, 7)
            hsv = plsc.load_gather(hs_v, [si])
            hdv = plsc.load_gather(hd_v, [di])
            zz = hsv + hdv
            lg = jnp.maximum(zz, 0.2 * zz)
            exv = jnp.exp(lg)
            eid = base + i * 16 + lane
            exv = jnp.where(eid < E, exv, 0.0)
            ex_v[sl] = exv
            plsc.store_scatter(denp_v, [i * 16 + lane,
                                        lax.rem(di, jnp.int32(128))], exv)
            return 0
        lax.fori_loop(0, CH // 16, phase_a, 0)

        def phase_b(e, _):
            eb = jnp.full((16,), e, jnp.int32)
            exb = plsc.load_gather(ex_v, [eb])
            etb = plsc.load_gather(et_v, [eb]) * HH
            db = plsc.load_gather(dst_v, [eb])
            off = lax.rem(db, jnp.int32(2)) * HH
            offz = HH - off
            a0 = plsc.load_gather(ea_v, [eb, zeros16])
            a1 = plsc.load_gather(ea_v, [eb, zeros16 + 1])
            a2 = plsc.load_gather(ea_v, [eb, zeros16 + 2])
            a3 = plsc.load_gather(ea_v, [eb, zeros16 + 3])
            eo_acc = zeros16f
            for j in range(HH // 16):
                sl = pl.ds(j * 16, 16)
                colj = j * 16 + lane
                efj = plsc.load_gather(ett_v, [etb + colj])
                efj = (efj + a0 * wea_v[0, sl] + a1 * wea_v[1, sl]
                       + a2 * wea_v[2, sl] + a3 * wea_v[3, sl])
                rowj = plsc.load_gather(rows_v, [eb, coff + colj])
                mj = jnp.maximum(rowj + efj, 0.0)
                plsc.store_scatter(pay_v, [eb, off + colj], mj * exb)
                plsc.store_scatter(pay_v, [eb, offz + colj], zeros16f)
                eo_acc = eo_acc + mj * woe_v[sl]
            eo_cum = plsc.cumsum(eo_acc)
            plsc.store_scatter(eo_v, [eb], eo_cum, mask=(lane == 15))
            return 0
        lax.fori_loop(0, CH, phase_b, 0)

        pltpu.sync_copy(pay_v, numer_sh.at[idxn_v], add=True)
        pltpu.sync_copy(denp_v, den_sh.at[idxd_v], add=True)
        pltpu.sync_copy(eo_v, eo_out.at[pl.ds(c * EPAD + base, CH)])

        def clear_denp(i, _):
            di = dst_v[pl.ds(i * 16, 16)]
            plsc.store_scatter(denp_v, [i * 16 + lane,
                                        lax.rem(di, jnp.int32(128))],
                               zeros16f)
            return 0
        lax.fori_loop(0, CH // 16, clear_denp, 0)
        return 0
    lax.fori_loop(0, NCHUNK, chunk, 0)

    plsc.subcore_barrier()
    pltpu.sync_copy(numer_sh.at[pl.ds(rn0, NRN)],
                    numer_out.at[c, pl.ds(rn0, NRN)])

    @pl.when(s == 0)
    def _():
        pltpu.sync_copy(den_sh.at[pl.ds(0, VD)],
                        den_out.at[c, pl.ds(0, VD)])


_sc = pl.kernel(
    _sc_body,
    out_type=[
        jax.ShapeDtypeStruct((NC, VP, 128), jnp.float32),
        jax.ShapeDtypeStruct((NC, VD, 128), jnp.float32),
        jax.ShapeDtypeStruct((NC * EPAD,), jnp.float32),
    ],
    mesh=plsc.VectorSubcoreMesh(core_axis_name="c", subcore_axis_name="s"),
    compiler_params=pltpu.CompilerParams(needs_layout_passes=False),
    scratch_types=[
        pltpu.VMEM((N,), jnp.float32),
        pltpu.VMEM((N,), jnp.float32),
        pltpu.VMEM((ET * HH,), jnp.float32),
        pltpu.VMEM((DE, HH), jnp.float32),
        pltpu.VMEM((HH,), jnp.float32),
        pltpu.VMEM((CH,), jnp.int32),
        pltpu.VMEM((CH,), jnp.int32),
        pltpu.VMEM((CH,), jnp.int32),
        pltpu.VMEM((CH,), jnp.int32),
        pltpu.VMEM((CH,), jnp.int32),
        pltpu.VMEM((CH, DE), jnp.float32),
        pltpu.VMEM((CH, H), jnp.float32),
        pltpu.VMEM((CH, H), jnp.float32),
        pltpu.VMEM((CH, H), jnp.float32),
        pltpu.VMEM((CH,), jnp.float32),
        pltpu.VMEM((CH,), jnp.float32),
        pltpu.VMEM_SHARED((VP, 128), jnp.float32),
        pltpu.VMEM_SHARED((VD, 128), jnp.float32),
        pltpu.SemaphoreType.DMA,
    ],
)


@jax.jit
def _run(x, src, dst, nt, et, edge_attr,
         W_in, nt_emb, et_emb, W_ea, W_msg, a_src, a_dst,
         W_out_node, W_out_edge):
    xp = jnp.pad(x, ((0, NP_ - N), (0, 0)))
    oh = jax.nn.one_hot(nt, 8, dtype=jnp.float32)
    ohp = jnp.pad(oh, ((0, NP_ - N), (0, 0)))
    nt8 = jnp.pad(nt_emb, ((0, 2), (0, 0)))
    amat = jnp.concatenate(
        [a_src[:, None], a_dst[:, None], jnp.zeros((H, 6), jnp.float32)],
        axis=1)
    h, hm, hsd = _tc1(xp, ohp, W_in, nt8, W_msg, amat)
    hs = hsd[:N, 0]
    hd = hsd[:N, 1]

    pe = EPAD - E
    srcp = jnp.pad(src, (0, pe))
    dstp = jnp.pad(dst, (0, pe))
    etp = jnp.pad(et, (0, pe))
    eap = jnp.pad(edge_attr, ((0, pe), (0, 0)))

    ett2 = jnp.stack([et_emb[:, :HH].reshape(-1), et_emb[:, HH:].reshape(-1)])
    wea2 = jnp.stack([W_ea[:, :HH], W_ea[:, HH:]])
    woe2 = jnp.stack([W_out_edge[:HH, 0], W_out_edge[HH:, 0]])

    numer, den, eo2 = _sc(
        hm, hs, hd, srcp, dstp, etp, eap, ett2, wea2, woe2,
        jnp.zeros((VP, 128), jnp.float32))

    numer_n = jnp.concatenate(
        [numer[0].reshape(NP_, HH), numer[1].reshape(NP_, HH)], axis=1)
    den_n = den[0].reshape(NP_, 1)

    wo8 = jnp.pad(W_out_node, ((0, 0), (0, 7)))
    no8 = _tc2(h, numer_n, den_n, wo8)
    eo = _tc3(eo2.reshape(NC, EPAD // 128, 128))
    return no8[:N, :1], eo.reshape(-1)[:E].reshape(E, 1)


def kernel(x, edge_index, node_type, edge_type, edge_attr,
           W_in, nt_emb, et_emb, W_ea, W_msg, a_src, a_dst,
           W_out_node, W_out_edge):
    src = edge_index[0].astype(jnp.int32)
    dst = edge_index[1].astype(jnp.int32)
    nt = node_type.astype(jnp.int32)
    et = edge_type.astype(jnp.int32)
    return _run(x, src, dst, nt, et, edge_attr,
                W_in, nt_emb, et_emb, W_ea, W_msg, a_src, a_dst,
                W_out_node, W_out_edge)


# speedup vs baseline: 3.8808x; 1.3357x over previous
import jax
import jax.numpy as jnp
from jax import lax
from jax.experimental import pallas as pl
from jax.experimental.pallas import tpu as pltpu
from jax.experimental.pallas import tpu_sc as plsc

N = 10000
E = 320000
D = 128
H = 128
NT = 6
ET = 4
DE = 4

NP_ = 10240
B1 = 256
NC = 2
NS = 16
HH = H // NC
CH = 64
EPT = 20096
EPAD = NS * EPT
NCHUNK = EPT // CH
VP = NP_ // 2
VD = NP_ // 128
NRN = VP // NS
NRD = VD // NS


def _tc1_body(x_ref, oh_ref, win_ref, nt8_ref, wmsg_ref, a_ref,
              h_ref, hm_ref, hsd_ref):
    xb = x_ref[...]
    h = jnp.maximum(
        jnp.dot(xb, win_ref[...], preferred_element_type=jnp.float32)
        + jnp.dot(oh_ref[...], nt8_ref[...],
                  preferred_element_type=jnp.float32), 0.0)
    h_ref[...] = h
    hm_ref[...] = jnp.dot(h, wmsg_ref[...],
                          preferred_element_type=jnp.float32)
    hsd_ref[...] = jnp.dot(h, a_ref[...],
                           preferred_element_type=jnp.float32)


_tc1 = pl.pallas_call(
    _tc1_body,
    grid=(NP_ // B1,),
    in_specs=[
        pl.BlockSpec((B1, D), lambda i: (i, 0)),
        pl.BlockSpec((B1, 8), lambda i: (i, 0)),
        pl.BlockSpec((D, H), lambda i: (0, 0)),
        pl.BlockSpec((8, H), lambda i: (0, 0)),
        pl.BlockSpec((H, H), lambda i: (0, 0)),
        pl.BlockSpec((H, 8), lambda i: (0, 0)),
    ],
    out_specs=[
        pl.BlockSpec((B1, H), lambda i: (i, 0)),
        pl.BlockSpec((B1, H), lambda i: (i, 0)),
        pl.BlockSpec((B1, 8), lambda i: (i, 0)),
    ],
    out_shape=[
        jax.ShapeDtypeStruct((NP_, H), jnp.float32),
        jax.ShapeDtypeStruct((NP_, H), jnp.float32),
        jax.ShapeDtypeStruct((NP_, 8), jnp.float32),
    ],
)


def _tc2_body(h_ref, num_ref, den_ref, wo_ref, out_ref):
    d = den_ref[...] + 1e-9
    agg = num_ref[...] / d
    h_out = jnp.maximum(h_ref[...] + agg, 0.0)
    out_ref[...] = jnp.dot(h_out, wo_ref[...],
                           preferred_element_type=jnp.float32)


_tc2 = pl.pallas_call(
    _tc2_body,
    grid=(NP_ // B1,),
    in_specs=[
        pl.BlockSpec((B1, H), lambda i: (i, 0)),
        pl.BlockSpec((B1, H), lambda i: (i, 0)),
        pl.BlockSpec((B1, 1), lambda i: (i, 0)),
        pl.BlockSpec((H, 8), lambda i: (0, 0)),
    ],
    out_specs=pl.BlockSpec((B1, 8), lambda i: (i, 0)),
    out_shape=jax.ShapeDtypeStruct((NP_, 8), jnp.float32),
)


def _tc3_body(eo_ref, out_ref):
    v = eo_ref[...]
    out_ref[...] = v[0] + v[1]


_tc3 = pl.pallas_call(
    _tc3_body,
    grid=(1,),
    in_specs=[pl.BlockSpec((NC, EPAD // 128, 128), lambda i: (0, 0, 0))],
    out_specs=pl.BlockSpec((EPAD // 128, 128), lambda i: (0, 0)),
    out_shape=jax.ShapeDtypeStruct((EPAD // 128, 128), jnp.float32),
)


def _sc_body(hm, hs, hd, srcp, dstp, etp, eap, ett2, wea2, woe2, z,
             numer_out, den_out, eo_out,
             hs_v, hd_v, ett_v, wea_v, woe_v,
             src_v0, dst_v0, et_v0, ea_v0, rows_v0,
             src_v1, dst_v1, et_v1, ea_v1, rows_v1,
             idxn_v, idxd_v, dcol_v, pay_v, denp_v, ex_v, eo_v,
             numer_sh, den_sh,
             sem_i0, sem_i1, sem_g0, sem_g1, sem_s, sem_e):
    c = lax.axis_index("c")
    s = lax.axis_index("s")

    pltpu.sync_copy(hs, hs_v)
    pltpu.sync_copy(hd, hd_v)
    pltpu.sync_copy(ett2.at[c], ett_v)
    pltpu.sync_copy(wea2.at[c], wea_v)
    pltpu.sync_copy(woe2.at[c], woe_v)

    rn0 = s * NRN
    pltpu.sync_copy(z.at[pl.ds(rn0, NRN)], numer_sh.at[pl.ds(rn0, NRN)])

    @pl.when(s == 0)
    def _():
        pltpu.sync_copy(z.at[pl.ds(0, VD)], den_sh.at[pl.ds(0, VD)])

    lane = lax.iota(jnp.int32, 16)
    zeros16 = jnp.zeros((16,), jnp.int32)
    zeros16f = jnp.zeros((16,), jnp.float32)

    def zero_denp(i, _):
        denp_v[i // 8, pl.ds((i % 8) * 16, 16)] = zeros16f
        return 0
    lax.fori_loop(0, CH * 8, zero_denp, 0)

    plsc.subcore_barrier()

    base0 = s * EPT
    coff = c * HH
    eobase = c * EPAD + base0

    bufs = ((src_v0, dst_v0, et_v0, ea_v0, rows_v0, sem_i0, sem_g0),
            (src_v1, dst_v1, et_v1, ea_v1, rows_v1, sem_i1, sem_g1))

    def start_in(g, b):
        base = base0 + g * CH
        sv, dv, ev, av, rv, si, sg = bufs[b]
        pltpu.async_copy(srcp.at[pl.ds(base, CH)], sv, si)
        pltpu.async_copy(dstp.at[pl.ds(base, CH)], dv, si)
        pltpu.async_copy(etp.at[pl.ds(base, CH)], ev, si)
        pltpu.async_copy(eap.at[pl.ds(base, CH)], av, si)

    def wait_in(b):
        sv, dv, ev, av, rv, si, sg = bufs[b]
        pltpu.make_async_copy(srcp.at[pl.ds(base0, CH)], sv, si).wait()
        pltpu.make_async_copy(dstp.at[pl.ds(base0, CH)], dv, si).wait()
        pltpu.make_async_copy(etp.at[pl.ds(base0, CH)], ev, si).wait()
        pltpu.make_async_copy(eap.at[pl.ds(base0, CH)], av, si).wait()

    def start_gather(b):
        sv, dv, ev, av, rv, si, sg = bufs[b]
        pltpu.async_copy(hm.at[sv], rv, sg)

    def wait_gather(b):
        sv, dv, ev, av, rv, si, sg = bufs[b]
        pltpu.make_async_copy(hm.at[sv], rv, sg).wait()

    def issue_scatters(g):
        pltpu.async_copy(pay_v, numer_sh.at[idxn_v], sem_s, add=True)
        pltpu.async_copy(denp_v, den_sh.at[idxd_v], sem_s, add=True)
        pltpu.async_copy(eo_v, eo_out.at[pl.ds(eobase + g * CH, CH)], sem_e)

    def drain_scatters():
        pltpu.make_async_copy(pay_v, numer_sh.at[idxn_v], sem_s).wait()
        pltpu.make_async_copy(denp_v, den_sh.at[idxd_v], sem_s).wait()
        pltpu.make_async_copy(eo_v, eo_out.at[pl.ds(eobase, CH)],
                              sem_e).wait()

        def clear_denp(i, _):
            dc = dcol_v[pl.ds(i * 16, 16)]
            plsc.store_scatter(denp_v, [i * 16 + lane, dc], zeros16f)
            return 0
        lax.fori_loop(0, CH // 16, clear_denp, 0)

    def compute(g, b):
        sv, dv, ev, av, rv, si, sg = bufs[b]
        base = base0 + g * CH

        def phase_a(i, _):
            sl = pl.ds(i * 16, 16)
            si_ = sv[sl]
            di = dv[sl]
            idxn_v[sl] = lax.shift_right_logical(di, 1)
            idxd_v[sl] = lax.shift_right_logical(di, 7)
            dcol_v[sl] = lax.rem(di, jnp.int32(128))
            hsv = plsc.load_gather(hs_v, [si_])
            hdv = plsc.load_gather(hd_v, [di])
            zz = hsv + hdv
            lg = jnp.maximum(zz, 0.2 * zz)
            exv = jnp.exp(lg)
            eid = base + i * 16 + lane
            exv = jnp.where(eid < E, exv, 0.0)
            ex_v[sl] = exv
            plsc.store_scatter(denp_v, [i * 16 + lane,
                                        lax.rem(di, jnp.int32(128))], exv)
            return 0
        lax.fori_loop(0, CH // 16, phase_a, 0)

        def phase_b(e, _):
            eb = jnp.full((16,), e, jnp.int32)
            exb = plsc.load_gather(ex_v, [eb])
            etb = plsc.load_gather(ev, [eb]) * HH
            db = plsc.load_gather(dv, [eb])
            off = lax.rem(db, jnp.int32(2)) * HH
            offz = HH - off
            a0 = plsc.load_gather(av, [eb, zeros16])
            a1 = plsc.load_gather(av, [eb, zeros16 + 1])
            a2 = plsc.load_gather(av, [eb, zeros16 + 2])
            a3 = plsc.load_gather(av, [eb, zeros16 + 3])
            eo_acc = zeros16f
            for j in range(HH // 16):
                sl = pl.ds(j * 16, 16)
                colj = j * 16 + lane
                efj = plsc.load_gather(ett_v, [etb + colj])
                efj = (efj + a0 * wea_v[0, sl] + a1 * wea_v[1, sl]
                       + a2 * wea_v[2, sl] + a3 * wea_v[3, sl])
                rowj = plsc.load_gather(rv, [eb, coff + colj])
                mj = jnp.maximum(rowj + efj, 0.0)
                plsc.store_scatter(pay_v, [eb, off + colj], mj * exb)
                plsc.store_scatter(pay_v, [eb, offz + colj], zeros16f)
                eo_acc = eo_acc + mj * woe_v[sl]
            eo_cum = plsc.cumsum(eo_acc)
            plsc.store_scatter(eo_v, [eb], eo_cum, mask=(lane == 15))
            return 0
        lax.fori_loop(0, CH, phase_b, 0)

    start_in(0, 0)

    def pair(k, _):
        g0 = 2 * k
        g1 = g0 + 1
        wait_in(0)
        start_gather(0)
        start_in(g1, 1)

        @pl.when(k > 0)
        def _():
            drain_scatters()
        wait_gather(0)
        compute(g0, 0)
        issue_scatters(g0)
        wait_in(1)
        start_gather(1)

        @pl.when(g1 + 1 < NCHUNK)
        def _():
            start_in(g1 + 1, 0)
        drain_scatters()
        wait_gather(1)
        compute(g1, 1)
        issue_scatters(g1)
        return 0
    lax.fori_loop(0, NCHUNK // 2, pair, 0)
    drain_scatters()

    plsc.subcore_barrier()
    pltpu.sync_copy(numer_sh.at[pl.ds(rn0, NRN)],
                    numer_out.at[c, pl.ds(rn0, NRN)])

    @pl.when(s == 0)
    def _():
        pltpu.sync_copy(den_sh.at[pl.ds(0, VD)],
                        den_out.at[c, pl.ds(0, VD)])


_sc = pl.kernel(
    _sc_body,
    out_type=[
        jax.ShapeDtypeStruct((NC, VP, 128), jnp.float32),
        jax.ShapeDtypeStruct((NC, VD, 128), jnp.float32),
        jax.ShapeDtypeStruct((NC * EPAD,), jnp.float32),
    ],
    mesh=plsc.VectorSubcoreMesh(core_axis_name="c", subcore_axis_name="s"),
    compiler_params=pltpu.CompilerParams(needs_layout_passes=False),
    scratch_types=[
        pltpu.VMEM((N,), jnp.float32),
        pltpu.VMEM((N,), jnp.float32),
        pltpu.VMEM((ET * HH,), jnp.float32),
        pltpu.VMEM((DE, HH), jnp.float32),
        pltpu.VMEM((HH,), jnp.float32),
        pltpu.VMEM((CH,), jnp.int32),
        pltpu.VMEM((CH,), jnp.int32),
        pltpu.VMEM((CH,), jnp.int32),
        pltpu.VMEM((CH, DE), jnp.float32),
        pltpu.VMEM((CH, H), jnp.float32),
        pltpu.VMEM((CH,), jnp.int32),
        pltpu.VMEM((CH,), jnp.int32),
        pltpu.VMEM((CH,), jnp.int32),
        pltpu.VMEM((CH, DE), jnp.float32),
        pltpu.VMEM((CH, H), jnp.float32),
        pltpu.VMEM((CH,), jnp.int32),
        pltpu.VMEM((CH,), jnp.int32),
        pltpu.VMEM((CH,), jnp.int32),
        pltpu.VMEM((CH, H), jnp.float32),
        pltpu.VMEM((CH, H), jnp.float32),
        pltpu.VMEM((CH,), jnp.float32),
        pltpu.VMEM((CH,), jnp.float32),
        pltpu.VMEM_SHARED((VP, 128), jnp.float32),
        pltpu.VMEM_SHARED((VD, 128), jnp.float32),
        pltpu.SemaphoreType.DMA,
        pltpu.SemaphoreType.DMA,
        pltpu.SemaphoreType.DMA,
        pltpu.SemaphoreType.DMA,
        pltpu.SemaphoreType.DMA,
        pltpu.SemaphoreType.DMA,
    ],
)


@jax.jit
def _run(x, src, dst, nt, et, edge_attr,
         W_in, nt_emb, et_emb, W_ea, W_msg, a_src, a_dst,
         W_out_node, W_out_edge):
    xp = jnp.pad(x, ((0, NP_ - N), (0, 0)))
    oh = jax.nn.one_hot(nt, 8, dtype=jnp.float32)
    ohp = jnp.pad(oh, ((0, NP_ - N), (0, 0)))
    nt8 = jnp.pad(nt_emb, ((0, 2), (0, 0)))
    amat = jnp.concatenate(
        [a_src[:, None], a_dst[:, None], jnp.zeros((H, 6), jnp.float32)],
        axis=1)
    h, hm, hsd = _tc1(xp, ohp, W_in, nt8, W_msg, amat)
    hs = hsd[:N, 0]
    hd = hsd[:N, 1]

    pe = EPAD - E
    srcp = jnp.pad(src, (0, pe))
    dstp = jnp.pad(dst, (0, pe))
    etp = jnp.pad(et, (0, pe))
    eap = jnp.pad(edge_attr, ((0, pe), (0, 0)))

    ett2 = jnp.stack([et_emb[:, :HH].reshape(-1), et_emb[:, HH:].reshape(-1)])
    wea2 = jnp.stack([W_ea[:, :HH], W_ea[:, HH:]])
    woe2 = jnp.stack([W_out_edge[:HH, 0], W_out_edge[HH:, 0]])

    numer, den, eo2 = _sc(
        hm, hs, hd, srcp, dstp, etp, eap, ett2, wea2, woe2,
        jnp.zeros((VP, 128), jnp.float32))

    numer_n = jnp.concatenate(
        [numer[0].reshape(NP_, HH), numer[1].reshape(NP_, HH)], axis=1)
    den_n = den[0].reshape(NP_, 1)

    wo8 = jnp.pad(W_out_node, ((0, 0), (0, 7)))
    no8 = _tc2(h, numer_n, den_n, wo8)
    eo = _tc3(eo2.reshape(NC, EPAD // 128, 128))
    return no8[:N, :1], eo.reshape(-1)[:E].reshape(E, 1)


def kernel(x, edge_index, node_type, edge_type, edge_attr,
           W_in, nt_emb, et_emb, W_ea, W_msg, a_src, a_dst,
           W_out_node, W_out_edge):
    src = edge_index[0].astype(jnp.int32)
    dst = edge_index[1].astype(jnp.int32)
    nt = node_type.astype(jnp.int32)
    et = edge_type.astype(jnp.int32)
    return _run(x, src, dst, nt, et, edge_attr,
                W_in, nt_emb, et_emb, W_ea, W_msg, a_src, a_dst,
                W_out_node, W_out_edge)


# speedup vs baseline: 4.1973x; 1.0816x over previous
import jax
import jax.numpy as jnp
from jax import lax
from jax.experimental import pallas as pl
from jax.experimental.pallas import tpu as pltpu
from jax.experimental.pallas import tpu_sc as plsc

N = 10000
E = 320000
D = 128
H = 128
NT = 6
ET = 4
DE = 4

NP_ = 10240
B1 = 256
NC = 2
NS = 16
HH = H // NC
CH = 64
EPT = 20096
EPAD = NS * EPT
NCHUNK = EPT // CH
VP = NP_ // 2
VD = NP_ // 128
NRN = VP // NS
NRD = VD // NS


def _tc1_body(x_ref, oh_ref, win_ref, nt8_ref, wmsg_ref, a_ref,
              h_ref, hm_ref, hsd_ref):
    xb = x_ref[...]
    h = jnp.maximum(
        jnp.dot(xb, win_ref[...], preferred_element_type=jnp.float32)
        + jnp.dot(oh_ref[...], nt8_ref[...],
                  preferred_element_type=jnp.float32), 0.0)
    h_ref[...] = h
    hm_ref[...] = jnp.dot(h, wmsg_ref[...],
                          preferred_element_type=jnp.float32)
    hsd_ref[...] = jnp.dot(h, a_ref[...],
                           preferred_element_type=jnp.float32)


_tc1 = pl.pallas_call(
    _tc1_body,
    grid=(NP_ // B1,),
    in_specs=[
        pl.BlockSpec((B1, D), lambda i: (i, 0)),
        pl.BlockSpec((B1, 8), lambda i: (i, 0)),
        pl.BlockSpec((D, H), lambda i: (0, 0)),
        pl.BlockSpec((8, H), lambda i: (0, 0)),
        pl.BlockSpec((H, H), lambda i: (0, 0)),
        pl.BlockSpec((H, 8), lambda i: (0, 0)),
    ],
    out_specs=[
        pl.BlockSpec((B1, H), lambda i: (i, 0)),
        pl.BlockSpec((B1, H), lambda i: (i, 0)),
        pl.BlockSpec((B1, 8), lambda i: (i, 0)),
    ],
    out_shape=[
        jax.ShapeDtypeStruct((NP_, H), jnp.float32),
        jax.ShapeDtypeStruct((NP_, H), jnp.float32),
        jax.ShapeDtypeStruct((NP_, 8), jnp.float32),
    ],
)


def _tc2_body(h_ref, num_ref, den_ref, wo_ref, out_ref):
    d = den_ref[...] + 1e-9
    agg = num_ref[...] / d
    h_out = jnp.maximum(h_ref[...] + agg, 0.0)
    out_ref[...] = jnp.dot(h_out, wo_ref[...],
                           preferred_element_type=jnp.float32)


_tc2 = pl.pallas_call(
    _tc2_body,
    grid=(NP_ // B1,),
    in_specs=[
        pl.BlockSpec((B1, H), lambda i: (i, 0)),
        pl.BlockSpec((B1, H), lambda i: (i, 0)),
        pl.BlockSpec((B1, 1), lambda i: (i, 0)),
        pl.BlockSpec((H, 8), lambda i: (0, 0)),
    ],
    out_specs=pl.BlockSpec((B1, 8), lambda i: (i, 0)),
    out_shape=jax.ShapeDtypeStruct((NP_, 8), jnp.float32),
)


def _tc3_body(eo_ref, out_ref):
    v = eo_ref[...]
    out_ref[...] = v[0] + v[1]


_tc3 = pl.pallas_call(
    _tc3_body,
    grid=(1,),
    in_specs=[pl.BlockSpec((NC, EPAD // 128, 128), lambda i: (0, 0, 0))],
    out_specs=pl.BlockSpec((EPAD // 128, 128), lambda i: (0, 0)),
    out_shape=jax.ShapeDtypeStruct((EPAD // 128, 128), jnp.float32),
)


def _sc_body(hm, hs, hd, srcp, dstp, etp, eap, ett2, wea2, woe2, z,
             numer_out, den_out, eo_out,
             hs_v, hd_v, ett_v, wea_v, woe_v,
             src_v0, dst_v0, et_v0, ea_v0, rows_v0,
             src_v1, dst_v1, et_v1, ea_v1, rows_v1,
             idxn_v, idxd_v, dcol_v, pay_v, denp_v, ex_v, eo_v,
             numer_sh, den_sh,
             sem_i0, sem_i1, sem_g0, sem_g1, sem_s, sem_e):
    c = lax.axis_index("c")
    s = lax.axis_index("s")

    pltpu.sync_copy(hs, hs_v)
    pltpu.sync_copy(hd, hd_v)
    pltpu.sync_copy(ett2.at[c], ett_v)
    pltpu.sync_copy(wea2.at[c], wea_v)
    pltpu.sync_copy(woe2.at[c], woe_v)

    rn0 = s * NRN
    pltpu.sync_copy(z.at[pl.ds(rn0, NRN)], numer_sh.at[pl.ds(rn0, NRN)])

    @pl.when(s == 0)
    def _():
        pltpu.sync_copy(z.at[pl.ds(0, VD)], den_sh.at[pl.ds(0, VD)])

    lane = lax.iota(jnp.int32, 16)
    zeros16 = jnp.zeros((16,), jnp.int32)
    zeros16f = jnp.zeros((16,), jnp.float32)

    def zero_denp(i, _):
        denp_v[i // 8, pl.ds((i % 8) * 16, 16)] = zeros16f
        return 0
    lax.fori_loop(0, CH * 8, zero_denp, 0)

    plsc.subcore_barrier()

    base0 = s * EPT
    coff = c * HH
    eobase = c * EPAD + base0

    bufs = ((src_v0, dst_v0, et_v0, ea_v0, rows_v0, sem_i0, sem_g0),
            (src_v1, dst_v1, et_v1, ea_v1, rows_v1, sem_i1, sem_g1))

    def start_in(g, b):
        base = base0 + g * CH
        sv, dv, ev, av, rv, si, sg = bufs[b]
        pltpu.async_copy(srcp.at[pl.ds(base, CH)], sv, si)
        pltpu.async_copy(dstp.at[pl.ds(base, CH)], dv, si)
        pltpu.async_copy(etp.at[pl.ds(base, CH)], ev, si)
        pltpu.async_copy(eap.at[pl.ds(base, CH)], av, si)

    def wait_in(b):
        sv, dv, ev, av, rv, si, sg = bufs[b]
        pltpu.make_async_copy(srcp.at[pl.ds(base0, CH)], sv, si).wait()
        pltpu.make_async_copy(dstp.at[pl.ds(base0, CH)], dv, si).wait()
        pltpu.make_async_copy(etp.at[pl.ds(base0, CH)], ev, si).wait()
        pltpu.make_async_copy(eap.at[pl.ds(base0, CH)], av, si).wait()

    def start_gather(b):
        sv, dv, ev, av, rv, si, sg = bufs[b]
        pltpu.async_copy(hm.at[sv], rv, sg)

    def wait_gather(b):
        sv, dv, ev, av, rv, si, sg = bufs[b]
        pltpu.make_async_copy(hm.at[sv], rv, sg).wait()

    def issue_scatters(g):
        pltpu.async_copy(pay_v, numer_sh.at[idxn_v], sem_s, add=True)
        pltpu.async_copy(denp_v, den_sh.at[idxd_v], sem_s, add=True)
        pltpu.async_copy(eo_v, eo_out.at[pl.ds(eobase + g * CH, CH)], sem_e)

    def drain_scatters():
        pltpu.make_async_copy(pay_v, numer_sh.at[idxn_v], sem_s).wait()
        pltpu.make_async_copy(denp_v, den_sh.at[idxd_v], sem_s).wait()
        pltpu.make_async_copy(eo_v, eo_out.at[pl.ds(eobase, CH)],
                              sem_e).wait()

        def clear_denp(i, _):
            dc = dcol_v[pl.ds(i * 16, 16)]
            plsc.store_scatter(denp_v, [i * 16 + lane, dc], zeros16f)
            return 0
        lax.fori_loop(0, CH // 16, clear_denp, 0)

    def compute(g, b):
        sv, dv, ev, av, rv, si, sg = bufs[b]
        base = base0 + g * CH

        def phase_a(i, _):
            sl = pl.ds(i * 16, 16)
            si_ = sv[sl]
            di = dv[sl]
            idxn_v[sl] = lax.shift_right_logical(di, 1)
            idxd_v[sl] = lax.shift_right_logical(di, 7)
            dcol_v[sl] = lax.rem(di, jnp.int32(128))
            hsv = plsc.load_gather(hs_v, [si_])
            hdv = plsc.load_gather(hd_v, [di])
            zz = hsv + hdv
            lg = jnp.maximum(zz, 0.2 * zz)
            exv = jnp.exp(lg)
            eid = base + i * 16 + lane
            exv = jnp.where(eid < E, exv, 0.0)
            ex_v[sl] = exv
            plsc.store_scatter(denp_v, [i * 16 + lane,
                                        lax.rem(di, jnp.int32(128))], exv)
            return 0
        lax.fori_loop(0, CH // 16, phase_a, 0)

        def phase_b(e, _):
            eb = jnp.full((16,), e, jnp.int32)
            exb = plsc.load_gather(ex_v, [eb])
            etb = plsc.load_gather(ev, [eb]) * HH
            db = plsc.load_gather(dv, [eb])
            off = lax.rem(db, jnp.int32(2)) * HH
            offz = HH - off
            a0 = plsc.load_gather(av, [eb, zeros16])
            a1 = plsc.load_gather(av, [eb, zeros16 + 1])
            a2 = plsc.load_gather(av, [eb, zeros16 + 2])
            a3 = plsc.load_gather(av, [eb, zeros16 + 3])
            eo_acc = zeros16f
            for j in range(HH // 16):
                sl = pl.ds(j * 16, 16)
                colj = j * 16 + lane
                efj = plsc.load_gather(ett_v, [etb + colj])
                efj = (efj + a0 * wea_v[0, sl] + a1 * wea_v[1, sl]
                       + a2 * wea_v[2, sl] + a3 * wea_v[3, sl])
                rowj = plsc.load_gather(rv, [eb, coff + colj])
                mj = jnp.maximum(rowj + efj, 0.0)
                plsc.store_scatter(pay_v, [eb, off + colj], mj * exb)
                plsc.store_scatter(pay_v, [eb, offz + colj], zeros16f)
                eo_acc = eo_acc + mj * woe_v[sl]
            eo_cum = plsc.cumsum(eo_acc)
            plsc.store_scatter(eo_v, [eb], eo_cum, mask=(lane == 15))
            return 0
        lax.fori_loop(0, CH, phase_b, 0)

    start_in(0, 0)
    wait_in(0)
    start_gather(0)
    start_in(1, 1)

    def pair(k, _):
        g0 = 2 * k
        g1 = g0 + 1
        wait_in(1)
        start_gather(1)

        @pl.when(k > 0)
        def _():
            drain_scatters()
        wait_gather(0)
        compute(g0, 0)
        issue_scatters(g0)

        @pl.when(g0 + 2 < NCHUNK)
        def _():
            start_in(g0 + 2, 0)
        @pl.when(g1 + 1 < NCHUNK)
        def _():
            wait_in(0)
            start_gather(0)
        drain_scatters()
        wait_gather(1)
        compute(g1, 1)
        issue_scatters(g1)

        @pl.when(g1 + 2 < NCHUNK)
        def _():
            start_in(g1 + 2, 1)
        return 0
    lax.fori_loop(0, NCHUNK // 2, pair, 0)
    drain_scatters()

    plsc.subcore_barrier()
    pltpu.sync_copy(numer_sh.at[pl.ds(rn0, NRN)],
                    numer_out.at[c, pl.ds(rn0, NRN)])

    @pl.when(s == 0)
    def _():
        pltpu.sync_copy(den_sh.at[pl.ds(0, VD)],
                        den_out.at[c, pl.ds(0, VD)])


_sc = pl.kernel(
    _sc_body,
    out_type=[
        jax.ShapeDtypeStruct((NC, VP, 128), jnp.float32),
        jax.ShapeDtypeStruct((NC, VD, 128), jnp.float32),
        jax.ShapeDtypeStruct((NC * EPAD,), jnp.float32),
    ],
    mesh=plsc.VectorSubcoreMesh(core_axis_name="c", subcore_axis_name="s"),
    compiler_params=pltpu.CompilerParams(needs_layout_passes=False),
    scratch_types=[
        pltpu.VMEM((N,), jnp.float32),
        pltpu.VMEM((N,), jnp.float32),
        pltpu.VMEM((ET * HH,), jnp.float32),
        pltpu.VMEM((DE, HH), jnp.float32),
        pltpu.VMEM((HH,), jnp.float32),
        pltpu.VMEM((CH,), jnp.int32),
        pltpu.VMEM((CH,), jnp.int32),
        pltpu.VMEM((CH,), jnp.int32),
        pltpu.VMEM((CH, DE), jnp.float32),
        pltpu.VMEM((CH, H), jnp.float32),
        pltpu.VMEM((CH,), jnp.int32),
        pltpu.VMEM((CH,), jnp.int32),
        pltpu.VMEM((CH,), jnp.int32),
        pltpu.VMEM((CH, DE), jnp.float32),
        pltpu.VMEM((CH, H), jnp.float32),
        pltpu.VMEM((CH,), jnp.int32),
        pltpu.VMEM((CH,), jnp.int32),
        pltpu.VMEM((CH,), jnp.int32),
        pltpu.VMEM((CH, H), jnp.float32),
        pltpu.VMEM((CH, H), jnp.float32),
        pltpu.VMEM((CH,), jnp.float32),
        pltpu.VMEM((CH,), jnp.float32),
        pltpu.VMEM_SHARED((VP, 128), jnp.float32),
        pltpu.VMEM_SHARED((VD, 128), jnp.float32),
        pltpu.SemaphoreType.DMA,
        pltpu.SemaphoreType.DMA,
        pltpu.SemaphoreType.DMA,
        pltpu.SemaphoreType.DMA,
        pltpu.SemaphoreType.DMA,
        pltpu.SemaphoreType.DMA,
    ],
)


@jax.jit
def _run(x, src, dst, nt, et, edge_attr,
         W_in, nt_emb, et_emb, W_ea, W_msg, a_src, a_dst,
         W_out_node, W_out_edge):
    xp = jnp.pad(x, ((0, NP_ - N), (0, 0)))
    oh = jax.nn.one_hot(nt, 8, dtype=jnp.float32)
    ohp = jnp.pad(oh, ((0, NP_ - N), (0, 0)))
    nt8 = jnp.pad(nt_emb, ((0, 2), (0, 0)))
    amat = jnp.concatenate(
        [a_src[:, None], a_dst[:, None], jnp.zeros((H, 6), jnp.float32)],
        axis=1)
    h, hm, hsd = _tc1(xp, ohp, W_in, nt8, W_msg, amat)
    hs = hsd[:N, 0]
    hd = hsd[:N, 1]

    pe = EPAD - E
    srcp = jnp.pad(src, (0, pe))
    dstp = jnp.pad(dst, (0, pe))
    etp = jnp.pad(et, (0, pe))
    eap = jnp.pad(edge_attr, ((0, pe), (0, 0)))

    ett2 = jnp.stack([et_emb[:, :HH].reshape(-1), et_emb[:, HH:].reshape(-1)])
    wea2 = jnp.stack([W_ea[:, :HH], W_ea[:, HH:]])
    woe2 = jnp.stack([W_out_edge[:HH, 0], W_out_edge[HH:, 0]])

    numer, den, eo2 = _sc(
        hm, hs, hd, srcp, dstp, etp, eap, ett2, wea2, woe2,
        jnp.zeros((VP, 128), jnp.float32))

    numer_n = jnp.concatenate(
        [numer[0].reshape(NP_, HH), numer[1].reshape(NP_, HH)], axis=1)
    den_n = den[0].reshape(NP_, 1)

    wo8 = jnp.pad(W_out_node, ((0, 0), (0, 7)))
    no8 = _tc2(h, numer_n, den_n, wo8)
    eo = _tc3(eo2.reshape(NC, EPAD // 128, 128))
    return no8[:N, :1], eo.reshape(-1)[:E].reshape(E, 1)


def kernel(x, edge_index, node_type, edge_type, edge_attr,
           W_in, nt_emb, et_emb, W_ea, W_msg, a_src, a_dst,
           W_out_node, W_out_edge):
    src = edge_index[0].astype(jnp.int32)
    dst = edge_index[1].astype(jnp.int32)
    nt = node_type.astype(jnp.int32)
    et = edge_type.astype(jnp.int32)
    return _run(x, src, dst, nt, et, edge_attr,
                W_in, nt_emb, et_emb, W_ea, W_msg, a_src, a_dst,
                W_out_node, W_out_edge)


# speedup vs baseline: 4.1977x; 1.0001x over previous
import jax
import jax.numpy as jnp
from jax import lax
from jax.experimental import pallas as pl
from jax.experimental.pallas import tpu as pltpu
from jax.experimental.pallas import tpu_sc as plsc

N = 10000
E = 320000
D = 128
H = 128
NT = 6
ET = 4
DE = 4

NP_ = 10240
B1 = 256
NC = 2
NS = 16
HH = H // NC
CH = 64
EPT = 20096
EPAD = NS * EPT
NCHUNK = EPT // CH
VP = NP_ // 2
VD = NP_ // 128
NRN = VP // NS
NRD = VD // NS


def _tc1_body(x_ref, oh_ref, win_ref, nt8_ref, wmsg_ref, a_ref,
              h_ref, hm_ref, hsd_ref):
    xb = x_ref[...]
    h = jnp.maximum(
        jnp.dot(xb, win_ref[...], preferred_element_type=jnp.float32)
        + jnp.dot(oh_ref[...], nt8_ref[...],
                  preferred_element_type=jnp.float32), 0.0)
    h_ref[...] = h
    hm_ref[...] = jnp.dot(h, wmsg_ref[...],
                          preferred_element_type=jnp.float32)
    hsd_ref[...] = jnp.dot(h, a_ref[...],
                           preferred_element_type=jnp.float32)


_tc1 = pl.pallas_call(
    _tc1_body,
    grid=(NP_ // B1,),
    in_specs=[
        pl.BlockSpec((B1, D), lambda i: (i, 0)),
        pl.BlockSpec((B1, 8), lambda i: (i, 0)),
        pl.BlockSpec((D, H), lambda i: (0, 0)),
        pl.BlockSpec((8, H), lambda i: (0, 0)),
        pl.BlockSpec((H, H), lambda i: (0, 0)),
        pl.BlockSpec((H, 8), lambda i: (0, 0)),
    ],
    out_specs=[
        pl.BlockSpec((B1, H), lambda i: (i, 0)),
        pl.BlockSpec((B1, H), lambda i: (i, 0)),
        pl.BlockSpec((B1, 8), lambda i: (i, 0)),
    ],
    out_shape=[
        jax.ShapeDtypeStruct((NP_, H), jnp.float32),
        jax.ShapeDtypeStruct((NP_, H), jnp.float32),
        jax.ShapeDtypeStruct((NP_, 8), jnp.float32),
    ],
)


def _tc2_body(h_ref, num_ref, den_ref, wo_ref, out_ref):
    d = den_ref[...] + 1e-9
    agg = num_ref[...] / d
    h_out = jnp.maximum(h_ref[...] + agg, 0.0)
    out_ref[...] = jnp.dot(h_out, wo_ref[...],
                           preferred_element_type=jnp.float32)


_tc2 = pl.pallas_call(
    _tc2_body,
    grid=(NP_ // B1,),
    in_specs=[
        pl.BlockSpec((B1, H), lambda i: (i, 0)),
        pl.BlockSpec((B1, H), lambda i: (i, 0)),
        pl.BlockSpec((B1, 1), lambda i: (i, 0)),
        pl.BlockSpec((H, 8), lambda i: (0, 0)),
    ],
    out_specs=pl.BlockSpec((B1, 8), lambda i: (i, 0)),
    out_shape=jax.ShapeDtypeStruct((NP_, 8), jnp.float32),
)


def _tc3_body(eo_ref, out_ref):
    v = eo_ref[...]
    out_ref[...] = v[0] + v[1]


_tc3 = pl.pallas_call(
    _tc3_body,
    grid=(1,),
    in_specs=[pl.BlockSpec((NC, EPAD // 128, 128), lambda i: (0, 0, 0))],
    out_specs=pl.BlockSpec((EPAD // 128, 128), lambda i: (0, 0)),
    out_shape=jax.ShapeDtypeStruct((EPAD // 128, 128), jnp.float32),
)


def _sc_body(hm, hs, hd, srcp, dstp, etp, eap, ett2, wea2, woe2, z,
             numer_out, den_out, eo_out,
             hs_v, hd_v, ett_v, wea_v, woe_v,
             src_v0, dst_v0, et_v0, ea_v0, rows_v0,
             src_v1, dst_v1, et_v1, ea_v1, rows_v1,
             idxn_v, idxd_v, dcol_v, pay_v, denp_v, ex_v, eo_v,
             numer_sh, den_sh,
             sem_i0, sem_i1, sem_g0, sem_g1, sem_s, sem_e):
    c = lax.axis_index("c")
    s = lax.axis_index("s")

    pltpu.sync_copy(hs, hs_v)
    pltpu.sync_copy(hd, hd_v)
    pltpu.sync_copy(ett2.at[c], ett_v)
    pltpu.sync_copy(wea2.at[c], wea_v)
    pltpu.sync_copy(woe2.at[c], woe_v)

    rn0 = s * NRN
    pltpu.sync_copy(z.at[pl.ds(rn0, NRN)], numer_sh.at[pl.ds(rn0, NRN)])

    @pl.when(s == 0)
    def _():
        pltpu.sync_copy(z.at[pl.ds(0, VD)], den_sh.at[pl.ds(0, VD)])

    lane = lax.iota(jnp.int32, 16)
    zeros16 = jnp.zeros((16,), jnp.int32)
    zeros16f = jnp.zeros((16,), jnp.float32)

    def zero_denp(i, _):
        denp_v[i // 8, pl.ds((i % 8) * 16, 16)] = zeros16f
        return 0
    lax.fori_loop(0, CH * 8, zero_denp, 0)

    plsc.subcore_barrier()

    base0 = s * EPT
    coff = c * HH
    eobase = c * EPAD + base0

    bufs = ((src_v0, dst_v0, et_v0, ea_v0, rows_v0, sem_i0, sem_g0),
            (src_v1, dst_v1, et_v1, ea_v1, rows_v1, sem_i1, sem_g1))

    def start_in(g, b):
        base = base0 + g * CH
        sv, dv, ev, av, rv, si, sg = bufs[b]
        pltpu.async_copy(srcp.at[pl.ds(base, CH)], sv, si)
        pltpu.async_copy(dstp.at[pl.ds(base, CH)], dv, si)
        pltpu.async_copy(etp.at[pl.ds(base, CH)], ev, si)
        pltpu.async_copy(eap.at[pl.ds(base, CH)], av, si)

    def wait_in(b):
        sv, dv, ev, av, rv, si, sg = bufs[b]
        pltpu.make_async_copy(srcp.at[pl.ds(base0, CH)], sv, si).wait()
        pltpu.make_async_copy(dstp.at[pl.ds(base0, CH)], dv, si).wait()
        pltpu.make_async_copy(etp.at[pl.ds(base0, CH)], ev, si).wait()
        pltpu.make_async_copy(eap.at[pl.ds(base0, CH)], av, si).wait()

    def start_gather(b):
        sv, dv, ev, av, rv, si, sg = bufs[b]
        pltpu.async_copy(hm.at[sv], rv, sg)

    def wait_gather(b):
        sv, dv, ev, av, rv, si, sg = bufs[b]
        pltpu.make_async_copy(hm.at[sv], rv, sg).wait()

    def issue_scatters(g):
        pltpu.async_copy(pay_v, numer_sh.at[idxn_v], sem_s, add=True)
        pltpu.async_copy(denp_v, den_sh.at[idxd_v], sem_s, add=True)
        pltpu.async_copy(eo_v, eo_out.at[pl.ds(eobase + g * CH, CH)], sem_e)

    def drain_scatters():
        pltpu.make_async_copy(pay_v, numer_sh.at[idxn_v], sem_s).wait()
        pltpu.make_async_copy(denp_v, den_sh.at[idxd_v], sem_s).wait()
        pltpu.make_async_copy(eo_v, eo_out.at[pl.ds(eobase, CH)],
                              sem_e).wait()

        def clear_denp(i, _):
            dc = dcol_v[pl.ds(i * 16, 16)]
            plsc.store_scatter(denp_v, [i * 16 + lane, dc], zeros16f)
            return 0
        lax.fori_loop(0, CH // 16, clear_denp, 0)

    def compute(g, b):
        sv, dv, ev, av, rv, si, sg = bufs[b]
        base = base0 + g * CH

        def phase_a(i, _):
            sl = pl.ds(i * 16, 16)
            si_ = sv[sl]
            di = dv[sl]
            idxn_v[sl] = lax.shift_right_logical(di, 1)
            idxd_v[sl] = lax.shift_right_logical(di, 7)
            dcol_v[sl] = lax.rem(di, jnp.int32(128))
            hsv = plsc.load_gather(hs_v, [si_])
            hdv = plsc.load_gather(hd_v, [di])
            zz = hsv + hdv
            lg = jnp.maximum(zz, 0.2 * zz)
            exv = jnp.exp(lg)
            eid = base + i * 16 + lane
            exv = jnp.where(eid < E, exv, 0.0)
            ex_v[sl] = exv
            plsc.store_scatter(denp_v, [i * 16 + lane,
                                        lax.rem(di, jnp.int32(128))], exv)
            return 0
        lax.fori_loop(0, CH // 16, phase_a, 0)

        def phase_b(e2, _):
            for u in range(2):
                e = e2 * 2 + u
                eb = jnp.full((16,), e, jnp.int32)
                exb = plsc.load_gather(ex_v, [eb])
                etb = plsc.load_gather(ev, [eb]) * HH
                db = plsc.load_gather(dv, [eb])
                off = lax.rem(db, jnp.int32(2)) * HH
                offz = HH - off
                a0 = plsc.load_gather(av, [eb, zeros16])
                a1 = plsc.load_gather(av, [eb, zeros16 + 1])
                a2 = plsc.load_gather(av, [eb, zeros16 + 2])
                a3 = plsc.load_gather(av, [eb, zeros16 + 3])
                eo_acc = zeros16f
                for j in range(HH // 16):
                    sl = pl.ds(j * 16, 16)
                    colj = j * 16 + lane
                    efj = plsc.load_gather(ett_v, [etb + colj])
                    efj = (efj + a0 * wea_v[0, sl] + a1 * wea_v[1, sl]
                           + a2 * wea_v[2, sl] + a3 * wea_v[3, sl])
                    rowj = plsc.load_gather(rv, [eb, coff + colj])
                    mj = jnp.maximum(rowj + efj, 0.0)
                    plsc.store_scatter(pay_v, [eb, off + colj], mj * exb)
                    plsc.store_scatter(pay_v, [eb, offz + colj], zeros16f)
                    eo_acc = eo_acc + mj * woe_v[sl]
                eo_cum = plsc.cumsum(eo_acc)
                plsc.store_scatter(eo_v, [eb], eo_cum, mask=(lane == 15))
            return 0
        lax.fori_loop(0, CH // 2, phase_b, 0)

    start_in(0, 0)
    wait_in(0)
    start_gather(0)
    start_in(1, 1)

    def pair(k, _):
        g0 = 2 * k
        g1 = g0 + 1
        wait_in(1)
        start_gather(1)

        @pl.when(k > 0)
        def _():
            drain_scatters()
        wait_gather(0)
        compute(g0, 0)
        issue_scatters(g0)

        @pl.when(g0 + 2 < NCHUNK)
        def _():
            start_in(g0 + 2, 0)
        @pl.when(g1 + 1 < NCHUNK)
        def _():
            wait_in(0)
            start_gather(0)
        drain_scatters()
        wait_gather(1)
        compute(g1, 1)
        issue_scatters(g1)

        @pl.when(g1 + 2 < NCHUNK)
        def _():
            start_in(g1 + 2, 1)
        return 0
    lax.fori_loop(0, NCHUNK // 2, pair, 0)
    drain_scatters()

    plsc.subcore_barrier()
    pltpu.sync_copy(numer_sh.at[pl.ds(rn0, NRN)],
                    numer_out.at[c, pl.ds(rn0, NRN)])

    @pl.when(s == 0)
    def _():
        pltpu.sync_copy(den_sh.at[pl.ds(0, VD)],
                        den_out.at[c, pl.ds(0, VD)])


_sc = pl.kernel(
    _sc_body,
    out_type=[
        jax.ShapeDtypeStruct((NC, VP, 128), jnp.float32),
        jax.ShapeDtypeStruct((NC, VD, 128), jnp.float32),
        jax.ShapeDtypeStruct((NC * EPAD,), jnp.float32),
    ],
    mesh=plsc.VectorSubcoreMesh(core_axis_name="c", subcore_axis_name="s"),
    compiler_params=pltpu.CompilerParams(needs_layout_passes=False),
    scratch_types=[
        pltpu.VMEM((N,), jnp.float32),
        pltpu.VMEM((N,), jnp.float32),
        pltpu.VMEM((ET * HH,), jnp.float32),
        pltpu.VMEM((DE, HH), jnp.float32),
        pltpu.VMEM((HH,), jnp.float32),
        pltpu.VMEM((CH,), jnp.int32),
        pltpu.VMEM((CH,), jnp.int32),
        pltpu.VMEM((CH,), jnp.int32),
        pltpu.VMEM((CH, DE), jnp.float32),
        pltpu.VMEM((CH, H), jnp.float32),
        pltpu.VMEM((CH,), jnp.int32),
        pltpu.VMEM((CH,), jnp.int32),
        pltpu.VMEM((CH,), jnp.int32),
        pltpu.VMEM((CH, DE), jnp.float32),
        pltpu.VMEM((CH, H), jnp.float32),
        pltpu.VMEM((CH,), jnp.int32),
        pltpu.VMEM((CH,), jnp.int32),
        pltpu.VMEM((CH,), jnp.int32),
        pltpu.VMEM((CH, H), jnp.float32),
        pltpu.VMEM((CH, H), jnp.float32),
        pltpu.VMEM((CH,), jnp.float32),
        pltpu.VMEM((CH,), jnp.float32),
        pltpu.VMEM_SHARED((VP, 128), jnp.float32),
        pltpu.VMEM_SHARED((VD, 128), jnp.float32),
        pltpu.SemaphoreType.DMA,
        pltpu.SemaphoreType.DMA,
        pltpu.SemaphoreType.DMA,
        pltpu.SemaphoreType.DMA,
        pltpu.SemaphoreType.DMA,
        pltpu.SemaphoreType.DMA,
    ],
)


@jax.jit
def _run(x, src, dst, nt, et, edge_attr,
         W_in, nt_emb, et_emb, W_ea, W_msg, a_src, a_dst,
         W_out_node, W_out_edge):
    xp = jnp.pad(x, ((0, NP_ - N), (0, 0)))
    oh = jax.nn.one_hot(nt, 8, dtype=jnp.float32)
    ohp = jnp.pad(oh, ((0, NP_ - N), (0, 0)))
    nt8 = jnp.pad(nt_emb, ((0, 2), (0, 0)))
    amat = jnp.concatenate(
        [a_src[:, None], a_dst[:, None], jnp.zeros((H, 6), jnp.float32)],
        axis=1)
    h, hm, hsd = _tc1(xp, ohp, W_in, nt8, W_msg, amat)
    hs = hsd[:N, 0]
    hd = hsd[:N, 1]

    pe = EPAD - E
    srcp = jnp.pad(src, (0, pe))
    dstp = jnp.pad(dst, (0, pe))
    etp = jnp.pad(et, (0, pe))
    eap = jnp.pad(edge_attr, ((0, pe), (0, 0)))

    ett2 = jnp.stack([et_emb[:, :HH].reshape(-1), et_emb[:, HH:].reshape(-1)])
    wea2 = jnp.stack([W_ea[:, :HH], W_ea[:, HH:]])
    woe2 = jnp.stack([W_out_edge[:HH, 0], W_out_edge[HH:, 0]])

    numer, den, eo2 = _sc(
        hm, hs, hd, srcp, dstp, etp, eap, ett2, wea2, woe2,
        jnp.zeros((VP, 128), jnp.float32))

    numer_n = jnp.concatenate(
        [numer[0].reshape(NP_, HH), numer[1].reshape(NP_, HH)], axis=1)
    den_n = den[0].reshape(NP_, 1)

    wo8 = jnp.pad(W_out_node, ((0, 0), (0, 7)))
    no8 = _tc2(h, numer_n, den_n, wo8)
    eo = _tc3(eo2.reshape(NC, EPAD // 128, 128))
    return no8[:N, :1], eo.reshape(-1)[:E].reshape(E, 1)


def kernel(x, edge_index, node_type, edge_type, edge_attr,
           W_in, nt_emb, et_emb, W_ea, W_msg, a_src, a_dst,
           W_out_node, W_out_edge):
    src = edge_index[0].astype(jnp.int32)
    dst = edge_index[1].astype(jnp.int32)
    nt = node_type.astype(jnp.int32)
    et = edge_type.astype(jnp.int32)
    return _run(x, src, dst, nt, et, edge_attr,
                W_in, nt_emb, et_emb, W_ea, W_msg, a_src, a_dst,
                W_out_node, W_out_edge)
